# Initial kernel scaffold; baseline (speedup 1.0000x reference)
import jax
import jax.numpy as jnp
from jax.experimental import pallas as pl


def kernel(x, edge_index, edge_attr, W_emb, b_emb, gat_Wl, gat_Wr, gat_We, gat_bl, gat_br, gat_bias, gat_att, ln_g, ln_b, pol_W1, pol_b1, pol_W2, pol_b2, pol_W3, pol_b3, val_W1, val_b1, val_W2, val_b2, coupling):
    raise NotImplementedError("write your pallas kernel here")


# speedup vs baseline: 1.0006x; 1.0006x over previous
import jax
import jax.numpy as jnp
from jax.experimental import pallas as pl

N = 512
E = 16384
D = 256
H = 8
C = 32
L = 4


def _layer_norm(x, g, b, eps=1e-5):
    mu = x.mean(axis=-1, keepdims=True)
    var = ((x - mu) ** 2).mean(axis=-1, keepdims=True)
    return (x - mu) / jnp.sqrt(var + eps) * g + b


def kernel(x, edge_index, edge_attr, W_emb, b_emb, gat_Wl, gat_Wr, gat_We, gat_bl, gat_br, gat_bias, gat_att, ln_g, ln_b, pol_W1, pol_b1, pol_W2, pol_b2, pol_W3, pol_b3, val_W1, val_b1, val_W2, val_b2, coupling):
    src = edge_index[0]
    dst = edge_index[1]
    n = x.shape[0]
    h = x @ W_emb + b_emb
    for i in range(L):
        res = h
        xl = (h @ gat_Wl[i] + gat_bl[i]).reshape(n, H, C)
        xr = (h @ gat_Wr[i] + gat_br[i]).reshape(n, H, C)
        ee = (edge_attr @ gat_We[i]).reshape(-1, H, C)
        m = xl[src] + xr[dst] + ee
        m = jnp.where(m > 0, m, 0.2 * m)
        alpha = (m * gat_att[i]).sum(-1)
        amax = jax.ops.segment_max(alpha, dst, num_segments=n)
        amax = jnp.where(jnp.isfinite(amax), amax, 0.0)
        ex = jnp.exp(alpha - amax[dst])
        den = jax.ops.segment_sum(ex, dst, num_segments=n)
        attn = ex / (den[dst] + 1e-16)
        out = jax.ops.segment_sum(attn[:, :, None] * xl[src], dst, num_segments=n)
        h2 = out.reshape(n, H * C) + gat_bias[i]
        h = _layer_norm(h2 + res, ln_g[i], ln_b[i])
        h = jax.nn.relu(h)
    graph_repr = jnp.concatenate([h.mean(axis=0, keepdims=True), h.max(axis=0, keepdims=True)], axis=1)
    ii, jj = jnp.triu_indices(n, k=1)
    er = jnp.concatenate([h[ii], h[jj]], axis=1)
    p = jax.nn.relu(er @ pol_W1 + pol_b1)
    p = jax.nn.relu(p @ pol_W2 + pol_b2)
    policy_logits = (p @ pol_W3 + pol_b3)[:, 0]
    v = jax.nn.relu(graph_repr @ val_W1 + val_b1)
    value = (v @ val_W2 + val_b2)[:, 0]
    colors = jnp.argmax(edge_attr, axis=1)
    energy = jnp.asarray(0.0, dtype=x.dtype)
    for c in (1, 2):
        w = (colors == c).astype(x.dtype)
        deg = jnp.zeros((n,), dtype=x.dtype).at[src].add(w).at[dst].add(w)
        energy = energy + (deg ** 2).sum() / (2.0 * n)
    energy = energy * coupling
    return (policy_logits, value, energy)


# speedup vs baseline: 5.8408x; 5.8371x over previous
import jax
import jax.numpy as jnp
from jax.experimental import pallas as pl
from jax.experimental.pallas import tpu as pltpu

_N = 512
_E = 16384
_D = 256
_H = 8
_C = 32
_L = 4
_EB = 1024
_NB = _E // _EB
_PBI = 64
_PBJ = 128
_HI = jax.lax.Precision.HIGHEST

_CP = pltpu.CompilerParams(vmem_limit_bytes=60 * 1024 * 1024)


def _embed_body(x_ref, w_ref, b_ref, o_ref):
    o_ref[...] = x_ref[...] * w_ref[...] + b_ref[...]


def _layer_body(h_ref, Wl_ref, Wr_ref, bl_ref, br_ref, We_ref, Aflat_ref,
                R_ref, bias_ref, lng_ref, lnb_ref, dst_rows_ref,
                src_cols_ref, dst_cols_ref, ea_cols_ref, o_ref,
                xl_scr, xr_scr, alpha_scr, amax_scr, num_scr, den_scr):
    h = h_ref[...]
    xl_scr[...] = jnp.dot(h, Wl_ref[...], precision=_HI) + bl_ref[...]
    xr_scr[...] = jnp.dot(h, Wr_ref[...], precision=_HI) + br_ref[...]
    iota_r = jax.lax.broadcasted_iota(jnp.int32, (1, _N), 1)
    iota_c = jax.lax.broadcasted_iota(jnp.int32, (_N, 1), 0)
    Aflat = Aflat_ref[...]
    neg_inf = jnp.float32(-jnp.inf)
    amax_scr[...] = jnp.full((_H, _N), neg_inf, dtype=jnp.float32)

    for b in range(_NB):
        src_col = src_cols_ref[:, b:b + 1]
        dst_col = dst_cols_ref[:, b:b + 1]
        oh_src = (src_col == iota_r).astype(jnp.float32)
        mask_dst = dst_col == iota_r
        xls = jnp.dot(oh_src, xl_scr[...], precision=_HI)
        xrd = jnp.dot(mask_dst.astype(jnp.float32), xr_scr[...], precision=_HI)
        ee = (ea_cols_ref[:, b:b + 1] * We_ref[0:1, :]
              + ea_cols_ref[:, _NB + b:_NB + b + 1] * We_ref[1:2, :]
              + ea_cols_ref[:, 2 * _NB + b:2 * _NB + b + 1] * We_ref[2:3, :])
        m = xls + xrd + ee
        m = jnp.where(m > 0, m, 0.2 * m)
        alpha = jnp.dot(m, Aflat, precision=_HI)
        alpha_scr[:, b * _H:(b + 1) * _H] = alpha
        for hh in range(_H):
            masked = jnp.where(mask_dst, alpha[:, hh:hh + 1], neg_inf)
            amax_scr[hh:hh + 1, :] = jnp.maximum(
                amax_scr[hh:hh + 1, :],
                jnp.max(masked, axis=0, keepdims=True))

    am = amax_scr[...]
    amax_cols = jnp.where(jnp.isfinite(am), am, 0.0).T
    num_scr[...] = jnp.zeros((_N, _D), dtype=jnp.float32)
    den_scr[...] = jnp.zeros((_N, _H), dtype=jnp.float32)
    R = R_ref[...]
    for b in range(_NB):
        src_col = src_cols_ref[:, b:b + 1]
        dst_col = dst_cols_ref[:, b:b + 1]
        dst_row = dst_rows_ref[b:b + 1, :]
        oh_src = (src_col == iota_r).astype(jnp.float32)
        mask_dst = (dst_col == iota_r).astype(jnp.float32)
        oh_dst_t = (iota_c == dst_row).astype(jnp.float32)
        xls = jnp.dot(oh_src, xl_scr[...], precision=_HI)
        amax_e = jnp.dot(mask_dst, amax_cols, precision=_HI)
        ex = jnp.exp(alpha_scr[:, b * _H:(b + 1) * _H] - amax_e)
        exd = jnp.dot(ex, R, precision=_HI)
        num_scr[...] += jnp.dot(oh_dst_t, exd * xls, precision=_HI)
        den_scr[...] += jnp.dot(oh_dst_t, ex, precision=_HI)

    invd = jnp.dot(1.0 / (den_scr[...] + 1e-16), R, precision=_HI)
    h2 = num_scr[...] * invd + bias_ref[...]
    y = h2 + h
    mu = jnp.mean(y, axis=1, keepdims=True)
    var = jnp.mean((y - mu) ** 2, axis=1, keepdims=True)
    hn = (y - mu) / jnp.sqrt(var + 1e-5) * lng_ref[...] + lnb_ref[...]
    o_ref[...] = jnp.maximum(hn, 0.0)


def _prepol_body(h_ref, W1t_ref, W1b_ref, b1_ref, a_ref, b_ref):
    h = h_ref[...]
    a_ref[...] = jnp.dot(h, W1t_ref[...], precision=_HI) + b1_ref[...]
    b_ref[...] = jnp.dot(h, W1b_ref[...], precision=_HI)


def _policy_body(A_ref, B_ref, W2_ref, b2_ref, W3_ref, b3_ref, o_ref):
    i = pl.program_id(0)
    j = pl.program_id(1)

    @pl.when(_PBJ * j + (_PBJ - 1) > _PBI * i)
    def _():
        Ai = A_ref[...][:, None, :]
        Bj = B_ref[...][None, :, :]
        P = jnp.maximum(Ai + Bj, 0.0).reshape(_PBI * _PBJ, _D)
        Q = jnp.maximum(
            jnp.dot(P, W2_ref[...], precision=_HI) + b2_ref[...], 0.0)
        Q3 = Q.reshape(_PBI, _PBJ, _D // 2)
        z = jnp.sum(Q3 * W3_ref[...], axis=2) + b3_ref[0, 0]
        o_ref[...] = z


def _tail_body(h_ref, vW1_ref, vb1_ref, vW2_ref, vb2_ref, src_rows_ref,
               dst_rows_ref, ea_cols_ref, coup_ref, val_ref, en_ref):
    h = h_ref[...]
    gmean = jnp.mean(h, axis=0, keepdims=True)
    gmax = jnp.max(h, axis=0, keepdims=True)
    gr = jnp.concatenate([gmean, gmax], axis=1)
    v = jnp.maximum(jnp.dot(gr, vW1_ref[...], precision=_HI) + vb1_ref[...],
                    0.0)
    val_ref[...] = jnp.dot(v, vW2_ref[...], precision=_HI) + vb2_ref[...]

    iota_c = jax.lax.broadcasted_iota(jnp.int32, (_N, 1), 0)
    degs = jnp.zeros((_N, 2), dtype=jnp.float32)
    for b in range(_NB):
        ea0 = ea_cols_ref[:, b:b + 1]
        ea1 = ea_cols_ref[:, _NB + b:_NB + b + 1]
        ea2 = ea_cols_ref[:, 2 * _NB + b:2 * _NB + b + 1]
        w1 = ((ea1 > ea0) & (ea1 >= ea2)).astype(jnp.float32)
        w2 = ((ea2 > ea0) & (ea2 > ea1)).astype(jnp.float32)
        w12 = jnp.concatenate([w1, w2], axis=1)
        oh_s = (iota_c == src_rows_ref[b:b + 1, :]).astype(jnp.float32)
        oh_d = (iota_c == dst_rows_ref[b:b + 1, :]).astype(jnp.float32)
        degs = degs + jnp.dot(oh_s, w12, precision=_HI) \
                    + jnp.dot(oh_d, w12, precision=_HI)
    en = jnp.sum(degs * degs) / (2.0 * _N) * coup_ref[0, 0]
    en_ref[...] = en.reshape(1, 1)


def kernel(x, edge_index, edge_attr, W_emb, b_emb, gat_Wl, gat_Wr, gat_We,
           gat_bl, gat_br, gat_bias, gat_att, ln_g, ln_b, pol_W1, pol_b1,
           pol_W2, pol_b2, pol_W3, pol_b3, val_W1, val_b1, val_W2, val_b2,
           coupling):
    f32 = jnp.float32
    src = edge_index[0]
    dst = edge_index[1]
    src_rows = src.reshape(_NB, _EB)
    dst_rows = dst.reshape(_NB, _EB)
    src_cols = src_rows.T
    dst_cols = dst_rows.T
    ea_cols = edge_attr.reshape(_NB, _EB, 3).transpose(1, 2, 0) \
                       .reshape(_EB, 3 * _NB)
    Aflat = jnp.einsum('lhc,hg->lhcg', gat_att,
                       jnp.eye(_H, dtype=f32)).reshape(_L, _D, _H)
    R = jnp.kron(jnp.eye(_H, dtype=f32), jnp.ones((1, _C), dtype=f32))

    sds = jax.ShapeDtypeStruct
    h = pl.pallas_call(
        _embed_body,
        out_shape=sds((_N, _D), f32),
        compiler_params=_CP,
    )(x, W_emb.reshape(1, _D), b_emb.reshape(1, _D))

    layer_call = pl.pallas_call(
        _layer_body,
        out_shape=sds((_N, _D), f32),
        scratch_shapes=[
            pltpu.VMEM((_N, _D), f32),
            pltpu.VMEM((_N, _D), f32),
            pltpu.VMEM((_EB, _NB * _H), f32),
            pltpu.VMEM((_H, _N), f32),
            pltpu.VMEM((_N, _D), f32),
            pltpu.VMEM((_N, _H), f32),
        ],
        compiler_params=_CP,
    )
    for i in range(_L):
        h = layer_call(h, gat_Wl[i], gat_Wr[i], gat_bl[i].reshape(1, _D),
                       gat_br[i].reshape(1, _D), gat_We[i], Aflat[i], R,
                       gat_bias[i].reshape(1, _D), ln_g[i].reshape(1, _D),
                       ln_b[i].reshape(1, _D), dst_rows, src_cols, dst_cols,
                       ea_cols)

    A, B = pl.pallas_call(
        _prepol_body,
        out_shape=(sds((_N, _D), f32), sds((_N, _D), f32)),
        compiler_params=_CP,
    )(h, pol_W1[:_D], pol_W1[_D:], pol_b1.reshape(1, _D))

    logits_full = pl.pallas_call(
        _policy_body,
        grid=(_N // _PBI, _N // _PBJ),
        in_specs=[
            pl.BlockSpec((_PBI, _D), lambda i, j: (i, 0)),
            pl.BlockSpec((_PBJ, _D), lambda i, j: (j, 0)),
            pl.BlockSpec((_D, _D // 2), lambda i, j: (0, 0)),
            pl.BlockSpec((1, _D // 2), lambda i, j: (0, 0)),
            pl.BlockSpec((1, 1, _D // 2), lambda i, j: (0, 0, 0)),
            pl.BlockSpec((1, 1), lambda i, j: (0, 0)),
        ],
        out_specs=pl.BlockSpec((_PBI, _PBJ), lambda i, j: (i, j)),
        out_shape=sds((_N, _N), f32),
        compiler_params=_CP,
    )(A, B, pol_W2, pol_b2.reshape(1, _D // 2),
      pol_W3.reshape(1, 1, _D // 2), pol_b3.reshape(1, 1))

    value, energy = pl.pallas_call(
        _tail_body,
        out_shape=(sds((1, 1), f32), sds((1, 1), f32)),
        compiler_params=_CP,
    )(h, val_W1, val_b1.reshape(1, _D), val_W2, val_b2.reshape(1, 1),
      src_rows, dst_rows, ea_cols, coupling.reshape(1, 1))

    ii, jj = jnp.triu_indices(_N, k=1)
    policy_logits = logits_full[ii, jj]
    return (policy_logits, value.reshape(1), energy.reshape(()))
